# baseline (speedup 1.0000x reference)
import math

import jax
import jax.numpy as jnp
from jax import lax
from jax.experimental import pallas as pl
from jax.experimental.pallas import tpu as pltpu

B = 8
N = 1024
T_DIM = 128
HID = 128
DIMS = 64
VOCAB = 128
OUT = DIMS * VOCAB

_HIGH = lax.Precision.HIGHEST


def _gcn_kernel(times_ref, a_ref, w1_ref, b1_ref, w2_ref, b2_ref, pool_ref,
                eye_ref, w1bT_ref, vec_ref, w2T_ref, freqs_ref):
    b = pl.program_id(0)

    @pl.when((b == 0) | (b == B // 2))
    def _init():
        ri = lax.broadcasted_iota(jnp.int32, (N, N), 0)
        ci = lax.broadcasted_iota(jnp.int32, (N, N), 1)
        eye_ref[...] = jnp.where(ri == ci, 1.0, 0.0).astype(jnp.bfloat16)
        w1bT_ref[...] = jnp.swapaxes(w1_ref[1:T_DIM + 1, :], 0, 1)
        rows = jnp.concatenate([w1_ref[0:1, :], b1_ref[...], b2_ref[...]],
                               axis=0)
        vec_ref[...] = jnp.swapaxes(rows, 0, 1)
        w2T_ref[...] = jnp.swapaxes(w2_ref[...], 0, 1)
        kk = lax.broadcasted_iota(jnp.int32, (T_DIM // 2, 1), 0)
        freqs_ref[...] = jnp.exp((-math.log(10000.0) / (T_DIM // 2 - 1))
                                 * kk.astype(jnp.float32))

    w1r0 = vec_ref[:, 0:1]
    b1c = vec_ref[:, 1:2]
    b2c = vec_ref[:, 2:3]

    a = a_ref[0]
    abf = a.astype(jnp.bfloat16)
    mbf = jnp.maximum(abf, eye_ref[...])
    onesb = jnp.ones((1, N), jnp.bfloat16)
    nd = jnp.sum(a, axis=0, keepdims=True)
    deg = jnp.dot(onesb, mbf, preferred_element_type=jnp.float32)
    inv = lax.rsqrt(deg)

    s = jnp.concatenate([inv * nd, inv], axis=0)
    uv = jnp.dot(s.astype(jnp.bfloat16), mbf,
                 preferred_element_type=jnp.float32)
    u = uv[0:1] * inv
    v = uv[1:2] * inv

    t = times_ref[0, b]
    args = t * freqs_ref[...]
    temb = jnp.concatenate([jnp.sin(args), jnp.cos(args)], axis=0)
    c = jnp.dot(w1bT_ref[...], temb, preferred_element_type=jnp.float32,
                precision=_HIGH)

    h1 = jnp.maximum(w1r0 * u + c * v + b1c, 0.0)

    xw2 = jnp.dot(w2T_ref[...], h1,
                  preferred_element_type=jnp.float32)
    z = xw2 * inv
    agg2 = jnp.dot(z.astype(jnp.bfloat16), mbf,
                   preferred_element_type=jnp.float32)
    h2 = jnp.maximum(agg2 * inv + b2c, 0.0)
    pool = jnp.mean(h2, axis=1, keepdims=True)
    pool_ref[...] = jnp.swapaxes(pool, 0, 1)[None]


def _proj_kernel(p_ref, w3_ref, b3_ref, out_ref):
    out_ref[...] = (jnp.dot(p_ref[...], w3_ref[...],
                            preferred_element_type=jnp.float32)
                    + b3_ref[...])


def kernel(adj, times, W1, b1, W2, b2, W3, b3):
    times2 = times.reshape(1, B)
    b1r = b1.reshape(1, HID)
    b2r = b2.reshape(1, HID)
    b3r = b3.reshape(1, OUT)

    pooled = pl.pallas_call(
        _gcn_kernel,
        grid=(B,),
        in_specs=[
            pl.BlockSpec(memory_space=pltpu.SMEM),
            pl.BlockSpec((1, N, N), lambda b: (b, 0, 0)),
            pl.BlockSpec((T_DIM + 1, HID), lambda b: (0, 0)),
            pl.BlockSpec((1, HID), lambda b: (0, 0)),
            pl.BlockSpec((HID, HID), lambda b: (0, 0)),
            pl.BlockSpec((1, HID), lambda b: (0, 0)),
        ],
        out_specs=pl.BlockSpec((1, 1, HID), lambda b: (b, 0, 0)),
        out_shape=jax.ShapeDtypeStruct((B, 1, HID), jnp.float32),
        scratch_shapes=[
            pltpu.VMEM((N, N), jnp.bfloat16),
            pltpu.VMEM((HID, T_DIM), jnp.float32),
            pltpu.VMEM((HID, 3), jnp.float32),
            pltpu.VMEM((HID, HID), jnp.float32),
            pltpu.VMEM((T_DIM // 2, 1), jnp.float32),
        ],
        compiler_params=pltpu.CompilerParams(
            dimension_semantics=("parallel",)),
    )(times2, adj, W1, b1r, W2, b2r)

    out = pl.pallas_call(
        _proj_kernel,
        in_specs=[
            pl.BlockSpec((B, HID), lambda: (0, 0)),
            pl.BlockSpec((HID, OUT), lambda: (0, 0)),
            pl.BlockSpec((1, OUT), lambda: (0, 0)),
        ],
        out_specs=pl.BlockSpec((B, OUT), lambda: (0, 0)),
        out_shape=jax.ShapeDtypeStruct((B, OUT), jnp.float32),
    )(pooled.reshape(B, HID), W3, b3r)
    return out.reshape(B, DIMS, VOCAB)
